# baseline (device time: 51201 ns/iter reference)
import jax
import jax.numpy as jnp
from jax import lax
from jax.experimental import pallas as pl
from jax.experimental.pallas import tpu as pltpu

N_DEV = 4
B, SQ, SKV = 2, 512, 512
HQ, DH = 8, 64
D_MODEL = 768
BLK = 64
STRIDE = 4
ROWS = B * SQ
CH = ROWS // N_DEV


def kernel(x, Wq, K_ext, V_ext, Wo):
    my = lax.axis_index("i")
    K = lax.dynamic_slice_in_dim(K_ext, my * HQ, HQ, axis=2)
    V = lax.dynamic_slice_in_dim(V_ext, my * HQ, HQ, axis=2)
    K = K.astype(jnp.bfloat16)
    V = V.astype(jnp.bfloat16)
    xb = x.reshape(ROWS, D_MODEL).astype(jnp.bfloat16)
    wq = (Wq * 0.125).astype(jnp.bfloat16)
    wo = Wo.astype(jnp.bfloat16)

    def body(x_ref, wq_ref, k_ref, v_ref, wo_ref, out_ref,
             q_ref, w_buf, send_buf, rs_ref, ag_ref,
             rs_send, rs_recv, ag_send, ag_recv):
        my_pos = lax.axis_index("i")

        q_all = jnp.dot(x_ref[...], wq_ref[...],
                        preferred_element_type=jnp.float32)
        q_ref[...] = q_all.astype(jnp.bfloat16)

        w_buf[...] = jnp.zeros((CH, SKV), jnp.bfloat16)

        barrier = pltpu.get_barrier_semaphore()
        for d in range(1, N_DEV):
            pl.semaphore_signal(
                barrier, inc=1,
                device_id=((my_pos + d) % N_DEV,),
                device_id_type=pl.DeviceIdType.MESH,
            )
        pl.semaphore_wait(barrier, N_DEV - 1)

        own_pc = None
        for i in range(N_DEV):
            q_idx = (my_pos + 1 + i) % N_DEV
            b = q_idx // 2
            qm = q_ref[pl.ds(q_idx * CH, CH), :]
            k_b = k_ref[b]
            v_b = v_ref[b]
            heads = []
            for h in range(HQ):
                qh = qm[:, h * DH:(h + 1) * DH]
                kh = k_b[:, h, :]
                vh = v_b[:, h, :]
                s = lax.dot_general(qh, kh, (((1,), (1,)), ((), ())),
                                    preferred_element_type=jnp.float32)
                denoms = []
                for j in range(STRIDE):
                    r = slice(BLK * j, BLK * (j + 1))
                    cA = slice(BLK * j, BLK * (j + 1))
                    cB = slice(4 * BLK + BLK * j, 4 * BLK + BLK * (j + 1))
                    eA = jnp.exp(s[r, cA])
                    eB = jnp.exp(s[r, cB])
                    denoms.append(jnp.sum(eA, axis=1, keepdims=True)
                                  + jnp.sum(eB, axis=1, keepdims=True))
                    w_buf[r, cA] = eA.astype(jnp.bfloat16)
                    w_buf[r, cB] = eB.astype(jnp.bfloat16)
                ctx = jnp.dot(w_buf[...], vh,
                              preferred_element_type=jnp.float32)
                ctx = ctx / jnp.concatenate(denoms, axis=0)
                heads.append(ctx.astype(jnp.bfloat16))
            ctx_m = jnp.concatenate(heads, axis=1)
            pc = jnp.dot(ctx_m, wo_ref[...],
                         preferred_element_type=jnp.float32)
            if i < N_DEV - 1:
                send_buf[N_DEV - 1 - i] = pc.astype(jnp.bfloat16)
                rdma = pltpu.make_async_remote_copy(
                    src_ref=send_buf.at[N_DEV - 1 - i],
                    dst_ref=rs_ref.at[N_DEV - 1 - i],
                    send_sem=rs_send.at[N_DEV - 1 - i],
                    recv_sem=rs_recv.at[N_DEV - 1 - i],
                    device_id=(q_idx,),
                    device_id_type=pl.DeviceIdType.MESH,
                )
                rdma.start()
            else:
                own_pc = pc

        red = own_pc
        for o in range(1, N_DEV):
            recv = pltpu.make_async_remote_copy(
                src_ref=send_buf.at[o], dst_ref=rs_ref.at[o],
                send_sem=rs_send.at[o], recv_sem=rs_recv.at[o],
                device_id=(my_pos,), device_id_type=pl.DeviceIdType.MESH,
            )
            recv.wait_recv()
            red = red + rs_ref[o].astype(jnp.float32)

        out_ref[pl.ds(my_pos * CH, CH), :] = red
        ag_ref[0] = red.astype(jnp.bfloat16)

        ag_sends = []
        for d in range(1, N_DEV):
            rdma = pltpu.make_async_remote_copy(
                src_ref=ag_ref.at[0],
                dst_ref=ag_ref.at[N_DEV - d],
                send_sem=ag_send.at[d - 1],
                recv_sem=ag_recv.at[N_DEV - d],
                device_id=((my_pos + d) % N_DEV,),
                device_id_type=pl.DeviceIdType.MESH,
            )
            rdma.start()
            ag_sends.append(rdma)

        for o in range(1, N_DEV):
            recv = pltpu.make_async_remote_copy(
                src_ref=ag_ref.at[0], dst_ref=ag_ref.at[o],
                send_sem=ag_send.at[0], recv_sem=ag_recv.at[o],
                device_id=(my_pos,), device_id_type=pl.DeviceIdType.MESH,
            )
            recv.wait_recv()
            p = (my_pos + o) % N_DEV
            out_ref[pl.ds(p * CH, CH), :] = ag_ref[o].astype(jnp.float32)

        for o in range(1, N_DEV):
            rdma = pltpu.make_async_remote_copy(
                src_ref=send_buf.at[o], dst_ref=rs_ref.at[o],
                send_sem=rs_send.at[o], recv_sem=rs_recv.at[o],
                device_id=(my_pos,), device_id_type=pl.DeviceIdType.MESH,
            )
            rdma.wait_send()
        for rdma in ag_sends:
            rdma.wait_send()

    out = pl.pallas_call(
        body,
        out_shape=jax.ShapeDtypeStruct((ROWS, D_MODEL), jnp.float32),
        in_specs=[pl.BlockSpec(memory_space=pltpu.VMEM)] * 5,
        out_specs=pl.BlockSpec(memory_space=pltpu.VMEM),
        scratch_shapes=[
            pltpu.VMEM((ROWS, HQ * DH), jnp.bfloat16),
            pltpu.VMEM((CH, SKV), jnp.bfloat16),
            pltpu.VMEM((N_DEV, CH, D_MODEL), jnp.bfloat16),
            pltpu.VMEM((N_DEV, CH, D_MODEL), jnp.bfloat16),
            pltpu.VMEM((N_DEV, CH, D_MODEL), jnp.bfloat16),
            pltpu.SemaphoreType.DMA((N_DEV,)),
            pltpu.SemaphoreType.DMA((N_DEV,)),
            pltpu.SemaphoreType.DMA((N_DEV - 1,)),
            pltpu.SemaphoreType.DMA((N_DEV,)),
        ],
        compiler_params=pltpu.CompilerParams(collective_id=0),
    )(xb, wq, K, V, wo)
    return out.reshape(B, SQ, D_MODEL)
